# Initial kernel scaffold; baseline (speedup 1.0000x reference)
import jax
import jax.numpy as jnp
from jax.experimental import pallas as pl


def kernel(X, background_vector, train_X, train_y, apparent_power_list):
    raise NotImplementedError("write your pallas kernel here")


# speedup vs baseline: 2.0139x; 2.0139x over previous
import jax
import jax.numpy as jnp
from jax import lax
from jax.experimental import pallas as pl
from jax.experimental.pallas import tpu as pltpu
from jax.experimental.pallas import tpu_sc as plsc

FFT = 65536
SPEC_OFF = 2 * FFT
N_PEAKS = 10
N_NEI = 5
N_KNOWN = 10
BG_N = 25
N_TRAIN = 100000
FEAT_DIM = 20
FREQ_RES = 15.2587890625
HALF_SR = 1000000.0
THRESH = 10.0

NC, NS = 2, 16
NW = NC * NS
SPEC_PER = FFT // NW
SPEC_GROUPS = SPEC_PER // 16
ROWS_PER = 3126
LAST_START = N_TRAIN - ROWS_PER
GROUPS_B = (ROWS_PER + 15) // 16

NEGF = -3.0e38
BIGF = 3.0e38
IBIG = 2147483647


def _lanes():
    return lax.broadcasted_iota(jnp.int32, (16,), 0)


def _stage_a_body(bg_hbm, x_hbm, vals_out, idx_out, bgbuf, clean, ovals, oidx,
                  sem):
    c = lax.axis_index("c")
    s = lax.axis_index("s")
    wid = s * NC + c
    off = wid * SPEC_PER

    cps = [pltpu.async_copy(bg_hbm.at[pl.ds(r * FFT + off, SPEC_PER)],
                            bgbuf.at[pl.ds(r * SPEC_PER, SPEC_PER)], sem)
           for r in range(BG_N)]
    cps.append(pltpu.async_copy(x_hbm.at[pl.ds(SPEC_OFF + off, SPEC_PER)],
                                clean, sem))
    for cp in cps:
        cp.wait()

    def bg_body(g, carry):
        base = g * 16
        parts = []
        for a in range(5):
            p = bgbuf[pl.ds(5 * a * SPEC_PER + base, 16)]
            for b in range(1, 5):
                p = p + bgbuf[pl.ds((5 * a + b) * SPEC_PER + base, 16)]
            parts.append(p)
        tot = ((parts[0] + parts[1]) + (parts[2] + parts[3])) + parts[4]
        slc = pl.ds(base, 16)
        clean[slc] = clean[slc] - tot / 25.0
        return carry

    lax.fori_loop(0, SPEC_GROUPS, bg_body, 0)

    li = _lanes()
    vacc = jnp.full((16,), NEGF, jnp.float32)
    iacc = jnp.full((16,), IBIG, jnp.int32)
    for j in range(N_PEAKS):
        def scan_body(g, carry):
            bv, bi = carry
            x = clean[pl.ds(g * 16, 16)]
            gi = g * 16 + li
            take = x > bv
            return jnp.where(take, x, bv), jnp.where(take, gi, bi)

        bv, bi = lax.fori_loop(
            0, SPEC_GROUPS, scan_body,
            (jnp.full((16,), NEGF, jnp.float32),
             jnp.full((16,), IBIG, jnp.int32)))
        vstar = jnp.max(bv)
        istar = jnp.min(jnp.where(bv == vstar, bi, IBIG))
        vacc = jnp.where(li == j, vstar, vacc)
        iacc = jnp.where(li == j, istar + off, iacc)
        plsc.store_scatter(clean, [jnp.full((16,), istar, jnp.int32)],
                           jnp.full((16,), NEGF, jnp.float32), mask=li == 0)

    ovals[...] = vacc
    oidx[...] = iacc
    pltpu.sync_copy(ovals, vals_out.at[pl.ds(wid * 16, 16)])
    pltpu.sync_copy(oidx, idx_out.at[pl.ds(wid * 16, 16)])


def _stage_b_body(cv_hbm, ci_hbm, tx_hbm, vals2_out, idx2_out,
                  cvals, cidx, xbuf, distb, ovals, oidx, sem):
    c = lax.axis_index("c")
    s = lax.axis_index("s")
    wid = s * NC + c
    row0 = jnp.where(wid == NW - 1, LAST_START, wid * ROWS_PER)

    cp1 = pltpu.async_copy(cv_hbm, cvals, sem)
    cp2 = pltpu.async_copy(ci_hbm, cidx, sem)
    cp3 = pltpu.async_copy(tx_hbm.at[pl.ds(row0 * FEAT_DIM,
                                           ROWS_PER * FEAT_DIM)], xbuf, sem)
    cp1.wait()
    cp2.wait()

    li = _lanes()
    fvacc = jnp.zeros((16,), jnp.float32)
    fiacc = jnp.zeros((16,), jnp.int32)
    for j in range(N_PEAKS):
        def mbody(t, carry):
            bv, bi, bp = carry
            x = cvals[pl.ds(t * 16, 16)]
            gi = cidx[pl.ds(t * 16, 16)]
            gp = t * 16 + li
            take = (x > bv) | ((x == bv) & (gi < bi))
            return (jnp.where(take, x, bv), jnp.where(take, gi, bi),
                    jnp.where(take, gp, bp))

        bv, bi, bp = lax.fori_loop(
            0, NW, mbody,
            (jnp.full((16,), NEGF, jnp.float32),
             jnp.full((16,), IBIG, jnp.int32),
             jnp.full((16,), IBIG, jnp.int32)))
        vstar = jnp.max(bv)
        m1 = bv == vstar
        istar = jnp.min(jnp.where(m1, bi, IBIG))
        pstar = jnp.min(jnp.where(m1 & (bi == istar), bp, IBIG))
        fvacc = jnp.where(li == j, vstar, fvacc)
        fiacc = jnp.where(li == j, istar, fiacc)
        plsc.store_scatter(cvals, [jnp.full((16,), pstar, jnp.int32)],
                           jnp.full((16,), NEGF, jnp.float32), mask=li == 0)

    ffreq = (fiacc.astype(jnp.float32) * FREQ_RES) / HALF_SR
    fs = []
    for k in range(N_PEAKS):
        fs.append(jnp.sum(jnp.where(li == k, fvacc, 0.0)))
    for k in range(N_PEAKS):
        fs.append(jnp.sum(jnp.where(li == k, ffreq, 0.0)))

    cp3.wait()

    def dbody(g, carry):
        rb = g * 16
        ri = rb + li
        valid = ri < ROWS_PER
        a0 = jnp.zeros((16,), jnp.float32)
        a1 = jnp.zeros((16,), jnp.float32)
        base = ri * FEAT_DIM
        for k in range(FEAT_DIM):
            xk = plsc.load_gather(xbuf, [base + k], mask=valid)
            d = xk - fs[k]
            if k % 2 == 0:
                a0 = a0 + d * d
            else:
                a1 = a1 + d * d
        distb[pl.ds(rb, 16)] = jnp.where(valid, a0 + a1, BIGF)
        return carry

    lax.fori_loop(0, GROUPS_B, dbody, 0)

    v5 = jnp.full((16,), BIGF, jnp.float32)
    i5 = jnp.full((16,), IBIG, jnp.int32)
    for j in range(N_NEI):
        def sbody(g, carry):
            bv, bi = carry
            x = distb[pl.ds(g * 16, 16)]
            gi = g * 16 + li
            take = x < bv
            return jnp.where(take, x, bv), jnp.where(take, gi, bi)

        bv, bi = lax.fori_loop(
            0, GROUPS_B, sbody,
            (jnp.full((16,), BIGF, jnp.float32),
             jnp.full((16,), IBIG, jnp.int32)))
        vstar = jnp.min(bv)
        istar = jnp.min(jnp.where(bv == vstar, bi, IBIG))
        v5 = jnp.where(li == j, vstar, v5)
        i5 = jnp.where(li == j, istar + row0, i5)
        plsc.store_scatter(distb, [jnp.full((16,), istar, jnp.int32)],
                           jnp.full((16,), BIGF, jnp.float32), mask=li == 0)

    ovals[...] = v5
    oidx[...] = i5
    pltpu.sync_copy(ovals, vals2_out.at[pl.ds(wid * 16, 16)])
    pltpu.sync_copy(oidx, idx2_out.at[pl.ds(wid * 16, 16)])


def _stage_c_body(v_ref, i_ref, pext_ref, ty_ref, out_ref, rowbuf, sem):
    d2 = v_ref[...]
    idx = i_ref[...]
    dist = jnp.sqrt(d2 + 1e-12)
    nn_d = []
    nn_i = []
    for j in range(N_NEI):
        vstar = jnp.min(dist)
        m = dist == vstar
        istar = jnp.min(jnp.where(m, idx, IBIG))
        nn_d.append(vstar)
        nn_i.append(istar)
        dist = jnp.where(m & (idx == istar), BIGF, dist)

    cps = []
    for j in range(N_NEI):
        cps.append(pltpu.make_async_copy(
            ty_ref.at[pl.ds(nn_i[j], 1), :], rowbuf.at[pl.ds(j, 1), :], sem))
    for cp in cps:
        cp.start()
    for cp in cps:
        cp.wait()

    rows = rowbuf[...]
    wv = jnp.stack([1.0 / (nn_d[j] + 1e-8) for j in range(N_NEI)])
    votes = jnp.sum(rows * wv[:, None], axis=0, keepdims=True)
    vmax = jnp.max(votes)
    lane11 = lax.broadcasted_iota(jnp.int32, (1, N_KNOWN + 1), 1)
    label = jnp.min(jnp.where(votes == vmax, lane11, jnp.int32(99)))
    label = jnp.where(nn_d[0] > THRESH, jnp.int32(N_KNOWN), label)
    lane16 = lax.broadcasted_iota(jnp.int32, (1, 16), 1)
    out_ref[...] = jnp.where(lane16 == label, pext_ref[...], 0.0)


_mesh = plsc.VectorSubcoreMesh(core_axis_name="c", subcore_axis_name="s",
                               num_cores=NC, num_subcores=NS)

_stage_a = pl.kernel(
    _stage_a_body,
    out_type=(jax.ShapeDtypeStruct((NW * 16,), jnp.float32),
              jax.ShapeDtypeStruct((NW * 16,), jnp.int32)),
    mesh=_mesh,
    scratch_types=[
        pltpu.VMEM((BG_N * SPEC_PER,), jnp.float32),
        pltpu.VMEM((SPEC_PER,), jnp.float32),
        pltpu.VMEM((16,), jnp.float32),
        pltpu.VMEM((16,), jnp.int32),
        pltpu.SemaphoreType.DMA,
    ],
    compiler_params=pltpu.CompilerParams(needs_layout_passes=False),
)

_stage_b = pl.kernel(
    _stage_b_body,
    out_type=(jax.ShapeDtypeStruct((NW * 16,), jnp.float32),
              jax.ShapeDtypeStruct((NW * 16,), jnp.int32)),
    mesh=_mesh,
    scratch_types=[
        pltpu.VMEM((NW * 16,), jnp.float32),
        pltpu.VMEM((NW * 16,), jnp.int32),
        pltpu.VMEM((ROWS_PER * FEAT_DIM,), jnp.float32),
        pltpu.VMEM((GROUPS_B * 16,), jnp.float32),
        pltpu.VMEM((16,), jnp.float32),
        pltpu.VMEM((16,), jnp.int32),
        pltpu.SemaphoreType.DMA,
    ],
    compiler_params=pltpu.CompilerParams(needs_layout_passes=False),
)


def _stage_c(v2, i2, pext16, train_y):
    return pl.pallas_call(
        _stage_c_body,
        out_shape=jax.ShapeDtypeStruct((1, 16), jnp.float32),
        in_specs=[
            pl.BlockSpec(memory_space=pltpu.MemorySpace.VMEM),
            pl.BlockSpec(memory_space=pltpu.MemorySpace.VMEM),
            pl.BlockSpec(memory_space=pltpu.MemorySpace.VMEM),
            pl.BlockSpec(memory_space=pl.ANY),
        ],
        out_specs=pl.BlockSpec(memory_space=pltpu.MemorySpace.VMEM),
        scratch_shapes=[
            pltpu.VMEM((N_NEI, N_KNOWN + 1), jnp.float32),
            pltpu.SemaphoreType.DMA,
        ],
    )(v2, i2, pext16, train_y)


def kernel(X, background_vector, train_X, train_y, apparent_power_list):
    cv, ci = _stage_a(background_vector.reshape(-1), X)
    v2, i2 = _stage_b(cv, ci, train_X.reshape(-1))
    pext = jnp.concatenate([apparent_power_list, X[3 * FFT:3 * FFT + 1]])
    pext16 = jnp.pad(pext, (0, 5)).reshape(1, 16)
    out = _stage_c(v2.reshape(NW, 16), i2.reshape(NW, 16), pext16, train_y)
    return out[0, :N_KNOWN + 1]
